# baseline (speedup 1.0000x reference)
import functools

import jax
import jax.numpy as jnp
from jax import lax
from jax.experimental import pallas as pl
from jax.experimental.pallas import tpu as pltpu
from jax.experimental.pallas import tpu_sc as plsc

_N_EMBD = 768
_BATCH = 32
_SEQ = 1024
_NC, _NS = 2, 16
_NW = _NC * _NS
_K = _SEQ // _NW
_L = 16
_PAIRS = _BATCH // 2
_G = 2 * _K
_RUNROLL = 4


def _emb_body(data_hbm, wte_hbm, wpe_hbm, out_hbm,
              idx_all, pos_v, bufs, gsems, osems):
    wid = lax.axis_index("s") * _NC + lax.axis_index("c")
    base_s = wid * _K

    pltpu.sync_copy(wpe_hbm.at[pl.ds(base_s, _K)], pos_v)

    pltpu.sync_copy(
        data_hbm.at[pl.ds(wid * (_BATCH * _K), _BATCH * _K)], idx_all)

    def gather(g, p):
        pltpu.async_copy(
            wte_hbm.at[idx_all.at[pl.ds(g * _G, _G)]], bufs[p], gsems[p])

    def gather_wait(g, p):
        pltpu.make_async_copy(
            wte_hbm.at[idx_all.at[pl.ds(g * _G, _G)]], bufs[p], gsems[p]).wait()

    def out_wait2(p):
        for _ in range(2):
            pltpu.make_async_copy(
                wte_hbm.at[pl.ds(0, _K)], bufs[p].at[pl.ds(0, _K)],
                osems[p]).wait()

    gather(0, 0)

    def step(i, _):
        for p in range(2):
            g = i * 2 + p
            q = 1 - p

            gather_wait(g, p)

            @pl.when(g + 1 < _PAIRS)
            def _():
                @pl.when(g >= 1)
                def _():
                    out_wait2(q)
                gather(g + 1, q)

            buf = bufs[p]

            def add_rows(r0, _):
                for dr in range(_RUNROLL):
                    r = r0 * _RUNROLL + dr
                    for j in range(_N_EMBD // _L):
                        sl = pl.ds(j * _L, _L)
                        x = pos_v[r, sl]
                        plsc.addupdate(buf.at[r, sl], x)
                        plsc.addupdate(buf.at[_K + r, sl], x)
                return 0

            lax.fori_loop(0, _K // _RUNROLL, add_rows, 0)
            pltpu.async_copy(
                buf.at[pl.ds(0, _K)],
                out_hbm.at[2 * g, pl.ds(base_s, _K)], osems[p])
            pltpu.async_copy(
                buf.at[pl.ds(_K, _K)],
                out_hbm.at[2 * g + 1, pl.ds(base_s, _K)], osems[p])
        return 0

    lax.fori_loop(0, _PAIRS // 2, step, 0)

    out_wait2(1)


@jax.jit
def kernel(data, wte, wpe):
    mesh = plsc.VectorSubcoreMesh(
        core_axis_name="c", subcore_axis_name="s",
        num_cores=_NC, num_subcores=_NS,
    )
    run = functools.partial(
        pl.kernel,
        out_type=jax.ShapeDtypeStruct((_BATCH, _SEQ, _N_EMBD), jnp.float32),
        mesh=mesh,
        scratch_types=[
            pltpu.VMEM((_BATCH * _K,), jnp.int32),
            pltpu.VMEM((_K, _N_EMBD), jnp.float32),
            tuple(pltpu.VMEM((_G, _N_EMBD), jnp.float32)
                  for _ in range(2)),
            tuple(pltpu.SemaphoreType.DMA for _ in range(2)),
            tuple(pltpu.SemaphoreType.DMA for _ in range(2)),
        ],
    )(_emb_body)
    data_t = data.reshape(_BATCH, _NW, _K).transpose(1, 0, 2).reshape(-1)
    return run(data_t, wte, wpe)


# speedup vs baseline: 1.1941x; 1.1941x over previous
import functools

import jax
import jax.numpy as jnp
from jax import lax
from jax.experimental import pallas as pl
from jax.experimental.pallas import tpu as pltpu
from jax.experimental.pallas import tpu_sc as plsc

_N_EMBD = 768
_BATCH = 32
_SEQ = 1024
_NC, _NS = 2, 16
_NW = _NC * _NS
_K = _SEQ // _NW
_L = 16
_NBUF = 4
_AHEAD = 3
_RUNROLL = 4


def _emb_body(data_hbm, wte_hbm, wpe_hbm, out_hbm,
              idx_all, pos_v, bufs, gsems, osems):
    wid = lax.axis_index("s") * _NC + lax.axis_index("c")
    base_s = wid * _K

    pltpu.sync_copy(wpe_hbm.at[pl.ds(base_s, _K)], pos_v)

    pltpu.sync_copy(
        data_hbm.at[pl.ds(wid * (_BATCH * _K), _BATCH * _K)], idx_all)

    def gather(b, p):
        pltpu.async_copy(
            wte_hbm.at[idx_all.at[pl.ds(b * _K, _K)]], bufs[p], gsems[p])

    def gather_wait(b, p):
        pltpu.make_async_copy(
            wte_hbm.at[idx_all.at[pl.ds(b * _K, _K)]], bufs[p], gsems[p]).wait()

    def out_wait(p):
        pltpu.make_async_copy(wte_hbm.at[pl.ds(0, _K)], bufs[p], osems[p]).wait()

    for b in range(_AHEAD):
        gather(b, b % _NBUF)

    def step(i, _):
        for p in range(_NBUF):
            b = i * _NBUF + p
            q = (p + _AHEAD) % _NBUF

            @pl.when(b + _AHEAD < _BATCH)
            def _():
                @pl.when(b >= _NBUF - _AHEAD)
                def _():
                    out_wait(q)
                gather(b + _AHEAD, q)

            gather_wait(b, p)
            buf = bufs[p]

            def add_rows(r0, _):
                for dr in range(_RUNROLL):
                    r = r0 * _RUNROLL + dr
                    for j in range(_N_EMBD // _L):
                        sl = pl.ds(j * _L, _L)
                        plsc.addupdate(buf.at[r, sl], pos_v[r, sl])
                return 0

            lax.fori_loop(0, _K // _RUNROLL, add_rows, 0)
            pltpu.async_copy(buf, out_hbm.at[b, pl.ds(base_s, _K)], osems[p])
        return 0

    lax.fori_loop(0, _BATCH // _NBUF, step, 0)

    for p in range(_NBUF):
        out_wait(p)


@jax.jit
def kernel(data, wte, wpe):
    mesh = plsc.VectorSubcoreMesh(
        core_axis_name="c", subcore_axis_name="s",
        num_cores=_NC, num_subcores=_NS,
    )
    run = functools.partial(
        pl.kernel,
        out_type=jax.ShapeDtypeStruct((_BATCH, _SEQ, _N_EMBD), jnp.float32),
        mesh=mesh,
        scratch_types=[
            pltpu.VMEM((_BATCH * _K,), jnp.int32),
            pltpu.VMEM((_K, _N_EMBD), jnp.float32),
            tuple(pltpu.VMEM((_K, _N_EMBD), jnp.float32)
                  for _ in range(_NBUF)),
            tuple(pltpu.SemaphoreType.DMA for _ in range(_NBUF)),
            tuple(pltpu.SemaphoreType.DMA for _ in range(_NBUF)),
        ],
    )(_emb_body)
    data_t = data.reshape(_BATCH, _NW, _K).transpose(1, 0, 2).reshape(-1)
    return run(data_t, wte, wpe)


# speedup vs baseline: 1.4116x; 1.1821x over previous
import functools

import jax
import jax.numpy as jnp
from jax import lax
from jax.experimental import pallas as pl
from jax.experimental.pallas import tpu as pltpu
from jax.experimental.pallas import tpu_sc as plsc

_N_EMBD = 768
_BATCH = 32
_SEQ = 1024
_NC, _NS = 2, 16
_NW = _NC * _NS
_K = _SEQ // _NW
_L = 16
_NBUF = 4
_AHEAD = 2
_RUNROLL = 4


def _emb_body(data_hbm, wte_hbm, wpe_hbm, out_hbm,
              idx_all, pos_v, bufs, gsems, osems):
    wid = lax.axis_index("s") * _NC + lax.axis_index("c")
    base_s = wid * _K

    pltpu.sync_copy(wpe_hbm.at[pl.ds(base_s, _K)], pos_v)

    pltpu.sync_copy(
        data_hbm.at[pl.ds(wid * (_BATCH * _K), _BATCH * _K)], idx_all)

    def gather(b, p):
        pltpu.async_copy(
            wte_hbm.at[idx_all.at[pl.ds(b * _K, _K)]], bufs[p], gsems[p])

    def gather_wait(b, p):
        pltpu.make_async_copy(
            wte_hbm.at[idx_all.at[pl.ds(b * _K, _K)]], bufs[p], gsems[p]).wait()

    def out_wait(p):
        pltpu.make_async_copy(wte_hbm.at[pl.ds(0, _K)], bufs[p], osems[p]).wait()

    for b in range(_AHEAD):
        gather(b, b % _NBUF)

    def step(i, _):
        for p in range(_NBUF):
            b = i * _NBUF + p
            q = (p + _AHEAD) % _NBUF

            @pl.when(b + _AHEAD < _BATCH)
            def _():
                @pl.when(b >= _NBUF - _AHEAD)
                def _():
                    out_wait(q)
                gather(b + _AHEAD, q)

            gather_wait(b, p)
            buf = bufs[p]

            def add_rows(r0, _):
                for dr in range(_RUNROLL):
                    r = r0 * _RUNROLL + dr
                    for j in range(_N_EMBD // _L):
                        sl = pl.ds(j * _L, _L)
                        plsc.addupdate(buf.at[r, sl], pos_v[r, sl])
                return 0

            lax.fori_loop(0, _K // _RUNROLL, add_rows, 0)
            pltpu.async_copy(buf, out_hbm.at[b, pl.ds(base_s, _K)], osems[p])
        return 0

    lax.fori_loop(0, _BATCH // _NBUF, step, 0)

    for p in range(_NBUF):
        out_wait(p)


@jax.jit
def kernel(data, wte, wpe):
    mesh = plsc.VectorSubcoreMesh(
        core_axis_name="c", subcore_axis_name="s",
        num_cores=_NC, num_subcores=_NS,
    )
    run = functools.partial(
        pl.kernel,
        out_type=jax.ShapeDtypeStruct((_BATCH, _SEQ, _N_EMBD), jnp.float32),
        mesh=mesh,
        scratch_types=[
            pltpu.VMEM((_BATCH * _K,), jnp.int32),
            pltpu.VMEM((_K, _N_EMBD), jnp.float32),
            tuple(pltpu.VMEM((_K, _N_EMBD), jnp.float32)
                  for _ in range(_NBUF)),
            tuple(pltpu.SemaphoreType.DMA for _ in range(_NBUF)),
            tuple(pltpu.SemaphoreType.DMA for _ in range(_NBUF)),
        ],
    )(_emb_body)
    data_t = data.reshape(_BATCH, _NW, _K).transpose(1, 0, 2).reshape(-1)
    return run(data_t, wte, wpe)
